# Initial kernel scaffold; baseline (speedup 1.0000x reference)
import jax
import jax.numpy as jnp
from jax.experimental import pallas as pl


def kernel(outputs, targets, targets_mask):
    raise NotImplementedError("write your pallas kernel here")


# speedup vs baseline: 31.3994x; 31.3994x over previous
import functools

import jax
import jax.numpy as jnp
from jax import lax
from jax.experimental import pallas as pl
from jax.experimental.pallas import tpu as pltpu

_K = 5
_NEG = -1e30


def _body(out_ref, tgt_ref, res_ref, acc_ref, num_ref, *, nsteps):
    i = pl.program_id(0)

    @pl.when(i == 0)
    def _init():
        acc_ref[0, 0] = jnp.float32(0.0)
        num_ref[0, 0] = jnp.float32(0.0)

    o = out_ref[...]
    t = tgt_ref[...]
    f32 = jnp.float32
    tf = t.astype(f32)
    F = o.shape[1]

    neg = t == 0
    m_neg = jnp.max(jnp.where(neg, o, _NEG), axis=1, keepdims=True)
    e = jnp.exp(o - m_neg)
    s_neg = jnp.sum(jnp.where(neg, e, f32(0.0)), axis=1, keepdims=True)

    gi = lax.broadcasted_iota(jnp.int32, (F, F), 0)
    fi = lax.broadcasted_iota(jnp.int32, (F, F), 1)
    tri = (gi < fi).astype(f32)
    csum = jnp.dot(tf, tri, preferred_element_type=f32)
    sel = jnp.logical_and(t == 1, csum < f32(_K) - f32(0.5))

    S = jnp.where(o <= m_neg, s_neg + e, s_neg / e + f32(1.0))
    Sm = jnp.where(sel, S, f32(1.0))
    p = Sm
    while p.shape[1] > 1:
        h = p.shape[1] // 2
        p = p[:, :h] * p[:, h:]
    prod = p[:, 0]
    relu = jnp.where(sel, jnp.maximum(m_neg - o, f32(0.0)), f32(0.0))
    part_loss = jnp.sum(jnp.log(prod)) + jnp.sum(relu)
    part_num = jnp.sum(tf)

    acc_ref[0, 0] += part_loss
    num_ref[0, 0] += part_num

    @pl.when(i == nsteps - 1)
    def _fin():
        num = num_ref[0, 0]
        res_ref[0, 0] = jnp.where(num > 0, acc_ref[0, 0] / num, jnp.float32(0.0))


def _run(outputs2, targets2, rows_per_block):
    n_rows = outputs2.shape[0]
    nsteps = n_rows // rows_per_block
    grid = (nsteps,)
    res = pl.pallas_call(
        functools.partial(_body, nsteps=nsteps),
        grid=grid,
        in_specs=[
            pl.BlockSpec((rows_per_block, outputs2.shape[1]), lambda i: (i, 0)),
            pl.BlockSpec((rows_per_block, outputs2.shape[1]), lambda i: (i, 0)),
        ],
        out_specs=pl.BlockSpec(memory_space=pltpu.SMEM),
        out_shape=jax.ShapeDtypeStruct((1, 1), jnp.float32),
        scratch_shapes=[
            pltpu.SMEM((1, 1), jnp.float32),
            pltpu.SMEM((1, 1), jnp.float32),
        ],
    )(outputs2, targets2)
    return res[0, 0]


def kernel(outputs, targets, targets_mask):
    B, T, F = targets.shape
    outputs2 = outputs.reshape(B * T, F)
    targets2 = targets.reshape(B * T, F)
    return _run(outputs2, targets2, rows_per_block=min(2048, B * T))
